# baseline (speedup 1.0000x reference)
import functools

import jax
import jax.numpy as jnp
from jax import lax
from jax.experimental import pallas as pl
from jax.experimental.pallas import tpu as pltpu
from jax.experimental.pallas import tpu_sc as plsc

BATCH = 4096
WINDOW = 50
EMBED = 128
TOTAL = BATCH * WINDOW
NUM_CORES = 2
NUM_SUBCORES = 16
NW = NUM_CORES * NUM_SUBCORES
CHUNK = 128
NCHUNK = WINDOW
NBUF = 6
AHEAD = 4


def _make_gather(vocab_size):
    mesh = plsc.VectorSubcoreMesh(core_axis_name="c", subcore_axis_name="s")

    @functools.partial(
        pl.kernel,
        mesh=mesh,
        out_type=jax.ShapeDtypeStruct((TOTAL, EMBED), jnp.float32),
        scratch_types=[
            pltpu.VMEM((NCHUNK, CHUNK), jnp.int32),
            pltpu.VMEM((NBUF, CHUNK, EMBED), jnp.float32),
            pltpu.SemaphoreType.DMA,
            pltpu.SemaphoreType.DMA,
        ],
    )
    def gather(idx_hbm, table_hbm, out_hbm, idx_v, bufs, gsem, ssem):
        wid = lax.axis_index("s") * NUM_CORES + lax.axis_index("c")
        col = wid * CHUNK
        pltpu.sync_copy(idx_hbm.at[pl.ds(0, 8), pl.ds(col, CHUNK)],
                        idx_v.at[pl.ds(0, 8)])

        def gcopy(i, slot):
            return pltpu.make_async_copy(
                table_hbm.at[idx_v.at[i]], bufs.at[slot], gsem)

        def scopy(i, slot):
            return pltpu.make_async_copy(
                bufs.at[slot], out_hbm.at[pl.ds(i * BATCH + col, CHUNK)], ssem)

        for i in range(AHEAD):
            gcopy(i, i).start()

        pltpu.sync_copy(idx_hbm.at[pl.ds(8, NCHUNK - 8), pl.ds(col, CHUNK)],
                        idx_v.at[pl.ds(8, NCHUNK - 8)])

        for i in range(NBUF - AHEAD):
            gcopy(i, i).wait()
            scopy(i, i).start()
            gcopy(i + AHEAD, i + AHEAD).start()

        def steady(i, carry):
            slot = lax.rem(i, NBUF)
            gcopy(i, slot).wait()
            scopy(i, slot).start()
            scopy(i, slot).wait()
            gcopy(i + AHEAD, lax.rem(i + AHEAD, NBUF)).start()
            return carry

        lax.fori_loop(NBUF - AHEAD, NCHUNK - AHEAD, steady, 0)

        for i in range(NCHUNK - AHEAD, NCHUNK):
            slot = i % NBUF
            gcopy(i, slot).wait()
            scopy(i, slot).start()
            scopy(i, slot).wait()

        for i in range(NBUF - AHEAD):
            scopy(NCHUNK - 1 - i, (NCHUNK - 1 - i) % NBUF).wait()

    return gather


def kernel(inputs, initial_state, embedding_table):
    out = _make_gather(embedding_table.shape[0])(inputs.T, embedding_table)
    return out.reshape(WINDOW, BATCH, EMBED).transpose(1, 0, 2)


# speedup vs baseline: 1.0046x; 1.0046x over previous
import functools

import jax
import jax.numpy as jnp
from jax import lax
from jax.experimental import pallas as pl
from jax.experimental.pallas import tpu as pltpu
from jax.experimental.pallas import tpu_sc as plsc

BATCH = 4096
WINDOW = 50
EMBED = 128
TOTAL = BATCH * WINDOW
NUM_CORES = 2
NUM_SUBCORES = 16
NW = NUM_CORES * NUM_SUBCORES
CHUNK = 128
NCHUNK = WINDOW
NBUF = 6
AHEAD = 3


def _make_gather(vocab_size):
    mesh = plsc.VectorSubcoreMesh(core_axis_name="c", subcore_axis_name="s")

    @functools.partial(
        pl.kernel,
        mesh=mesh,
        out_type=jax.ShapeDtypeStruct((TOTAL, EMBED), jnp.float32),
        scratch_types=[
            pltpu.VMEM((NCHUNK, CHUNK), jnp.int32),
            pltpu.VMEM((NBUF, CHUNK, EMBED), jnp.float32),
            pltpu.SemaphoreType.DMA,
            pltpu.SemaphoreType.DMA,
        ],
    )
    def gather(idx_hbm, table_hbm, out_hbm, idx_v, bufs, gsem, ssem):
        wid = lax.axis_index("s") * NUM_CORES + lax.axis_index("c")
        col = wid * CHUNK
        pltpu.sync_copy(idx_hbm.at[pl.ds(0, 8), pl.ds(col, CHUNK)],
                        idx_v.at[pl.ds(0, 8)])

        def gcopy(i, slot):
            return pltpu.make_async_copy(
                table_hbm.at[idx_v.at[i]], bufs.at[slot], gsem)

        def scopy(i, slot):
            return pltpu.make_async_copy(
                bufs.at[slot], out_hbm.at[pl.ds(i * BATCH + col, CHUNK)], ssem)

        for i in range(AHEAD):
            gcopy(i, i).start()

        pltpu.sync_copy(idx_hbm.at[pl.ds(8, NCHUNK - 8), pl.ds(col, CHUNK)],
                        idx_v.at[pl.ds(8, NCHUNK - 8)])

        for i in range(NBUF - AHEAD):
            gcopy(i, i).wait()
            scopy(i, i).start()
            gcopy(i + AHEAD, i + AHEAD).start()

        def steady(i, carry):
            slot = lax.rem(i, NBUF)
            gcopy(i, slot).wait()
            scopy(i, slot).start()
            scopy(i, slot).wait()
            gcopy(i + AHEAD, lax.rem(i + AHEAD, NBUF)).start()
            return carry

        lax.fori_loop(NBUF - AHEAD, NCHUNK - AHEAD, steady, 0)

        for i in range(NCHUNK - AHEAD, NCHUNK):
            slot = i % NBUF
            gcopy(i, slot).wait()
            scopy(i, slot).start()
            scopy(i, slot).wait()

        for i in range(NBUF - AHEAD):
            scopy(NCHUNK - 1 - i, (NCHUNK - 1 - i) % NBUF).wait()

    return gather


def kernel(inputs, initial_state, embedding_table):
    out = _make_gather(embedding_table.shape[0])(inputs.T, embedding_table)
    return out.reshape(WINDOW, BATCH, EMBED).transpose(1, 0, 2)
